# baseline (speedup 1.0000x reference)
import functools

import jax
import jax.numpy as jnp
from jax import lax
from jax.experimental import pallas as pl
from jax.experimental.pallas import tpu as pltpu
from jax.experimental.pallas import tpu_sc as plsc

B = 16
L = 512
T = B * L
V = 50265
D = 1024
P = 514
PAD = 1
EPS = 1e-5

NC = 2
NS = 16
NW = NC * NS
TPW = T // NW
K = 16
NCHUNK = TPW // K
LPS = L // 16


def _combine_tables(pos_emb, type_emb):
    def body(pos_ref, type_ref, out_ref):
        out_ref[...] = pos_ref[...] + type_ref[0, :][None, :]
    return pl.pallas_call(
        body,
        out_shape=jax.ShapeDtypeStruct((P, D), jnp.float32),
    )(pos_emb, type_emb)


def _sc_embed_body(ids_hbm, word_hbm, pt_hbm, lnw_hbm, lnb_hbm, out_hbm,
                   ids_v, pos_v, lnw_v, lnb_v, wbuf, pbuf, obuf):
    cid = lax.axis_index("c")
    sid = lax.axis_index("s")
    wid = sid * NC + cid
    seq = wid // 2
    half = wid % 2

    pltpu.sync_copy(ids_hbm.at[pl.ds(seq * L, L)], ids_v)
    pltpu.sync_copy(lnw_hbm, lnw_v)
    pltpu.sync_copy(lnb_hbm, lnb_v)

    def posloop(g, carry):
        vec = ids_v[pl.ds(g * 16, 16)]
        m = vec != PAD
        mi = jnp.where(m, jnp.int32(1), jnp.int32(0))
        cs = plsc.cumsum(mi) + carry
        pos_v[pl.ds(g * 16, 16)] = jnp.where(m, cs, jnp.int32(0)) + PAD
        return carry + jnp.sum(mi)
    lax.fori_loop(0, LPS, posloop, jnp.int32(0), unroll=False)

    half_base = half * TPW

    def chunk(c, _):
        base = half_base + c * K
        row0 = seq * L + base
        pltpu.sync_copy(word_hbm.at[ids_v.at[pl.ds(base, K)]], wbuf)
        pltpu.sync_copy(pt_hbm.at[pos_v.at[pl.ds(base, K)]], pbuf)
        for t in range(K):
            def p1(dd, acc):
                s, q = acc
                v = wbuf[t, pl.ds(dd * 16, 16)] + pbuf[t, pl.ds(dd * 16, 16)]
                wbuf[t, pl.ds(dd * 16, 16)] = v
                return (s + v, q + v * v)
            zero = jnp.zeros((16,), jnp.float32)
            s, q = lax.fori_loop(0, D // 16, p1, (zero, zero), unroll=4)
            mean = jnp.sum(s) * (1.0 / D)
            var = jnp.sum(q) * (1.0 / D) - mean * mean
            x = jnp.broadcast_to(var + EPS, (16,))
            iv = plsc.bitcast(x, jnp.int32)
            iv = jnp.int32(0x5F3759DF) - (iv >> 1)
            y = plsc.bitcast(iv, jnp.float32)
            for _ in range(3):
                y = y * (1.5 - 0.5 * x * y * y)
            mvec = jnp.broadcast_to(mean, (16,))

            def p2(dd, carry):
                v = wbuf[t, pl.ds(dd * 16, 16)]
                obuf[t, pl.ds(dd * 16, 16)] = (
                    (v - mvec) * y * lnw_v[pl.ds(dd * 16, 16)]
                    + lnb_v[pl.ds(dd * 16, 16)])
                return carry
            lax.fori_loop(0, D // 16, p2, 0, unroll=4)
        pltpu.sync_copy(obuf, out_hbm.at[pl.ds(row0, K)])
        return 0
    lax.fori_loop(0, NCHUNK, chunk, 0, unroll=False)


_sc_embed = functools.partial(
    pl.kernel,
    out_type=jax.ShapeDtypeStruct((T, D), jnp.float32),
    mesh=plsc.VectorSubcoreMesh(core_axis_name="c", subcore_axis_name="s"),
    compiler_params=pltpu.CompilerParams(needs_layout_passes=False),
    scratch_types=[
        pltpu.VMEM((L,), jnp.int32),
        pltpu.VMEM((L,), jnp.int32),
        pltpu.VMEM((D,), jnp.float32),
        pltpu.VMEM((D,), jnp.float32),
        pltpu.VMEM((K, D), jnp.float32),
        pltpu.VMEM((K, D), jnp.float32),
        pltpu.VMEM((K, D), jnp.float32),
    ],
)(_sc_embed_body)


def kernel(input_ids, seq_lens, position_ids, token_type_ids,
           word_emb, pos_emb, type_emb, ln_w, ln_b):
    pt = _combine_tables(pos_emb, type_emb)
    return _sc_embed(input_ids, word_emb, pt, ln_w, ln_b)


# speedup vs baseline: 1.1974x; 1.1974x over previous
import functools

import jax
import jax.numpy as jnp
from jax import lax
from jax.experimental import pallas as pl
from jax.experimental.pallas import tpu as pltpu
from jax.experimental.pallas import tpu_sc as plsc

B = 16
L = 512
T = B * L
V = 50265
D = 1024
P = 514
PAD = 1
EPS = 1e-5

NC = 2
NS = 16
NW = NC * NS
TPW = T // NW
K = 16
NCHUNK = TPW // K
LPS = L // 16


def _combine_tables(pos_emb, type_emb):
    def body(pos_ref, type_ref, out_ref):
        out_ref[...] = pos_ref[...] + type_ref[0, :][None, :]
    return pl.pallas_call(
        body,
        out_shape=jax.ShapeDtypeStruct((P, D), jnp.float32),
    )(pos_emb, type_emb)


def _sc_embed_body(ids_hbm, word_hbm, pt_hbm, lnw_hbm, lnb_hbm, out_hbm,
                   ids_v, pos_v, lnw_v, lnb_v,
                   wbuf0, wbuf1, pbuf0, pbuf1, obuf0, obuf1,
                   gw0, gw1, gp0, gp1, so0, so1):
    wbufs = (wbuf0, wbuf1)
    pbufs = (pbuf0, pbuf1)
    obufs = (obuf0, obuf1)
    gws = (gw0, gw1)
    gps = (gp0, gp1)
    sos = (so0, so1)

    cid = lax.axis_index("c")
    sid = lax.axis_index("s")
    wid = sid * NC + cid
    seq = wid // 2
    half = wid % 2

    pltpu.sync_copy(ids_hbm.at[pl.ds(seq * L, L)], ids_v)
    pltpu.sync_copy(lnw_hbm, lnw_v)
    pltpu.sync_copy(lnb_hbm, lnb_v)

    def posloop(g, carry):
        vec = ids_v[pl.ds(g * 16, 16)]
        m = vec != PAD
        mi = jnp.where(m, jnp.int32(1), jnp.int32(0))
        cs = plsc.cumsum(mi) + carry
        pos_v[pl.ds(g * 16, 16)] = jnp.where(m, cs, jnp.int32(0)) + PAD
        return carry + jnp.sum(mi)
    lax.fori_loop(0, LPS, posloop, jnp.int32(0), unroll=False)

    half_base = half * TPW

    def issue_gathers(c, b):
        base = half_base + c * K
        pltpu.async_copy(word_hbm.at[ids_v.at[pl.ds(base, K)]], wbufs[b], gws[b])
        pltpu.async_copy(pt_hbm.at[pos_v.at[pl.ds(base, K)]], pbufs[b], gps[b])

    for b in range(2):
        issue_gathers(jnp.int32(b), b)

    def do_chunk(c, b):
        wbuf, pbuf, obuf = wbufs[b], pbufs[b], obufs[b]
        pltpu.make_async_copy(word_hbm.at[pl.ds(0, K)], wbuf, gws[b]).wait()
        pltpu.make_async_copy(pt_hbm.at[pl.ds(0, K)], pbuf, gps[b]).wait()
        for t in range(K):
            def p1(dd, acc):
                s, q = acc
                v = wbuf[t, pl.ds(dd * 16, 16)] + pbuf[t, pl.ds(dd * 16, 16)]
                wbuf[t, pl.ds(dd * 16, 16)] = v
                return (s + v, q + v * v)
            zero = jnp.zeros((16,), jnp.float32)
            s, q = lax.fori_loop(0, D // 16, p1, (zero, zero), unroll=4)
            mean = jnp.sum(s) * (1.0 / D)
            var = jnp.sum(q) * (1.0 / D) - mean * mean
            x = jnp.broadcast_to(var + EPS, (16,))
            iv = plsc.bitcast(x, jnp.int32)
            iv = jnp.int32(0x5F3759DF) - (iv >> 1)
            y = plsc.bitcast(iv, jnp.float32)
            for _ in range(3):
                y = y * (1.5 - 0.5 * x * y * y)
            mvec = jnp.broadcast_to(mean, (16,))

            def p2(dd, carry):
                v = wbuf[t, pl.ds(dd * 16, 16)]
                obuf[t, pl.ds(dd * 16, 16)] = (
                    (v - mvec) * y * lnw_v[pl.ds(dd * 16, 16)]
                    + lnb_v[pl.ds(dd * 16, 16)])
                return carry
            lax.fori_loop(0, D // 16, p2, 0, unroll=4)
        row0 = seq * L + half_base + c * K
        pltpu.async_copy(obuf, out_hbm.at[pl.ds(row0, K)], sos[b])

    def pairloop(i, _):
        for b in range(2):
            c = 2 * i + b

            @pl.when(i > 0)
            def _wait_out():
                pltpu.make_async_copy(
                    obufs[b], out_hbm.at[pl.ds(0, K)], sos[b]).wait()

            do_chunk(c, b)

            @pl.when(c + 2 < NCHUNK)
            def _next_gather():
                issue_gathers(c + 2, b)
        return 0
    lax.fori_loop(0, NCHUNK // 2, pairloop, 0, unroll=False)
    for b in range(2):
        pltpu.make_async_copy(obufs[b], out_hbm.at[pl.ds(0, K)], sos[b]).wait()


_sc_embed = functools.partial(
    pl.kernel,
    out_type=jax.ShapeDtypeStruct((T, D), jnp.float32),
    mesh=plsc.VectorSubcoreMesh(core_axis_name="c", subcore_axis_name="s"),
    compiler_params=pltpu.CompilerParams(needs_layout_passes=False),
    scratch_types=[
        pltpu.VMEM((L,), jnp.int32),
        pltpu.VMEM((L,), jnp.int32),
        pltpu.VMEM((D,), jnp.float32),
        pltpu.VMEM((D,), jnp.float32),
        pltpu.VMEM((K, D), jnp.float32),
        pltpu.VMEM((K, D), jnp.float32),
        pltpu.VMEM((K, D), jnp.float32),
        pltpu.VMEM((K, D), jnp.float32),
        pltpu.VMEM((K, D), jnp.float32),
        pltpu.VMEM((K, D), jnp.float32),
        pltpu.SemaphoreType.DMA,
        pltpu.SemaphoreType.DMA,
        pltpu.SemaphoreType.DMA,
        pltpu.SemaphoreType.DMA,
        pltpu.SemaphoreType.DMA,
        pltpu.SemaphoreType.DMA,
    ],
)(_sc_embed_body)


def kernel(input_ids, seq_lens, position_ids, token_type_ids,
           word_emb, pos_emb, type_emb, ln_w, ln_b):
    pt = _combine_tables(pos_emb, type_emb)
    return _sc_embed(input_ids, word_emb, pt, ln_w, ln_b)


# speedup vs baseline: 3.2884x; 2.7462x over previous
import functools

import jax
import jax.numpy as jnp
from jax import lax
from jax.experimental import pallas as pl
from jax.experimental.pallas import tpu as pltpu
from jax.experimental.pallas import tpu_sc as plsc

B = 16
L = 512
T = B * L
V = 50265
D = 1024
P = 514
PAD = 1
EPS = 1e-5

NC = 2
NS = 16
NW = NC * NS
TPW = T // NW
K = 16
NCHUNK = TPW // K
LPS = L // 16


def _combine_tables(pos_emb, type_emb):
    def body(pos_ref, type_ref, out_ref):
        out_ref[...] = pos_ref[...] + type_ref[0, :][None, :]
    return pl.pallas_call(
        body,
        out_shape=jax.ShapeDtypeStruct((P, D), jnp.float32),
    )(pos_emb, type_emb)


def _sc_embed_body(ids_hbm, word_hbm, pt_hbm, lnw_hbm, lnb_hbm, out_hbm,
                   ids_v, pos_v, lnw_v, lnb_v,
                   wbuf0, wbuf1, pbuf0, pbuf1, obuf0, obuf1,
                   gw0, gw1, gp0, gp1, so0, so1):
    wbufs = (wbuf0, wbuf1)
    pbufs = (pbuf0, pbuf1)
    obufs = (obuf0, obuf1)
    gws = (gw0, gw1)
    gps = (gp0, gp1)
    sos = (so0, so1)

    cid = lax.axis_index("c")
    sid = lax.axis_index("s")
    wid = sid * NC + cid
    seq = wid // 2
    half = wid % 2

    pltpu.sync_copy(ids_hbm.at[pl.ds(seq * L, L)], ids_v)
    pltpu.sync_copy(lnw_hbm, lnw_v)
    pltpu.sync_copy(lnb_hbm, lnb_v)

    def posloop(g, carry):
        vec = ids_v[pl.ds(g * 16, 16)]
        m = vec != PAD
        mi = jnp.where(m, jnp.int32(1), jnp.int32(0))
        cs = plsc.cumsum(mi) + carry
        pos_v[pl.ds(g * 16, 16)] = jnp.where(m, cs, jnp.int32(0)) + PAD
        return carry + jnp.sum(mi)
    lax.fori_loop(0, LPS, posloop, jnp.int32(0), unroll=False)

    half_base = half * TPW

    def issue_gathers(c, b):
        base = half_base + c * K
        pltpu.async_copy(word_hbm.at[ids_v.at[pl.ds(base, K)]], wbufs[b], gws[b])
        pltpu.async_copy(pt_hbm.at[pos_v.at[pl.ds(base, K)]], pbufs[b], gps[b])

    for b in range(2):
        issue_gathers(jnp.int32(b), b)

    def do_chunk(c, b):
        wbuf, pbuf, obuf = wbufs[b], pbufs[b], obufs[b]
        pltpu.make_async_copy(word_hbm.at[pl.ds(0, K)], wbuf, gws[b]).wait()
        pltpu.make_async_copy(pt_hbm.at[pl.ds(0, K)], pbuf, gps[b]).wait()
        for g in range(K // 8):
            stats = []
            for t8 in range(8):
                t = g * 8 + t8
                zero = jnp.zeros((16,), jnp.float32)

                @plsc.parallel_loop(0, D // 16, step=4, carry=(zero,) * 8)
                def p1(dd, acc, t=t):
                    accs = list(acc)
                    for u in range(4):
                        sl = pl.ds((dd + u) * 16, 16)
                        v = wbuf[t, sl] + pbuf[t, sl]
                        obuf[t, sl] = v
                        accs[2 * u] = accs[2 * u] + v
                        accs[2 * u + 1] = accs[2 * u + 1] + v * v
                    return tuple(accs)
                acc = p1
                s = (acc[0] + acc[2]) + (acc[4] + acc[6])
                q = (acc[1] + acc[3]) + (acc[5] + acc[7])
                mean = jnp.sum(s) * (1.0 / D)
                var = jnp.sum(q) * (1.0 / D) - mean * mean
                x = jnp.broadcast_to(var + EPS, (16,))
                iv = plsc.bitcast(x, jnp.int32)
                iv = jnp.int32(0x5F3759DF) - (iv >> 1)
                y = plsc.bitcast(iv, jnp.float32)
                for _ in range(3):
                    y = y * (1.5 - 0.5 * x * y * y)
                stats.append((y, jnp.broadcast_to(mean, (16,)) * y))

            @plsc.parallel_loop(0, D // 16)
            def p2(dd, g=g, stats=stats):
                sl = pl.ds(dd * 16, 16)
                wv = lnw_v[sl]
                bv = lnb_v[sl]
                for t8 in range(8):
                    t = g * 8 + t8
                    a, am = stats[t8]
                    v = obuf[t, sl]
                    obuf[t, sl] = (v * a - am) * wv + bv
        row0 = seq * L + half_base + c * K
        pltpu.async_copy(obuf, out_hbm.at[pl.ds(row0, K)], sos[b])

    def pairloop(i, _):
        for b in range(2):
            c = 2 * i + b

            @pl.when(i > 0)
            def _wait_out():
                pltpu.make_async_copy(
                    obufs[b], out_hbm.at[pl.ds(0, K)], sos[b]).wait()

            do_chunk(c, b)

            @pl.when(c + 2 < NCHUNK)
            def _next_gather():
                issue_gathers(c + 2, b)
        return 0
    lax.fori_loop(0, NCHUNK // 2, pairloop, 0, unroll=False)
    for b in range(2):
        pltpu.make_async_copy(obufs[b], out_hbm.at[pl.ds(0, K)], sos[b]).wait()


_sc_embed = functools.partial(
    pl.kernel,
    out_type=jax.ShapeDtypeStruct((T, D), jnp.float32),
    mesh=plsc.VectorSubcoreMesh(core_axis_name="c", subcore_axis_name="s"),
    compiler_params=pltpu.CompilerParams(needs_layout_passes=False),
    scratch_types=[
        pltpu.VMEM((L,), jnp.int32),
        pltpu.VMEM((L,), jnp.int32),
        pltpu.VMEM((D,), jnp.float32),
        pltpu.VMEM((D,), jnp.float32),
        pltpu.VMEM((K, D), jnp.float32),
        pltpu.VMEM((K, D), jnp.float32),
        pltpu.VMEM((K, D), jnp.float32),
        pltpu.VMEM((K, D), jnp.float32),
        pltpu.VMEM((K, D), jnp.float32),
        pltpu.VMEM((K, D), jnp.float32),
        pltpu.SemaphoreType.DMA,
        pltpu.SemaphoreType.DMA,
        pltpu.SemaphoreType.DMA,
        pltpu.SemaphoreType.DMA,
        pltpu.SemaphoreType.DMA,
        pltpu.SemaphoreType.DMA,
    ],
)(_sc_embed_body)


def kernel(input_ids, seq_lens, position_ids, token_type_ids,
           word_emb, pos_emb, type_emb, ln_w, ln_b):
    pt = _combine_tables(pos_emb, type_emb)
    return _sc_embed(input_ids, word_emb, pt, ln_w, ln_b)
